# baseline (speedup 1.0000x reference)
import functools

import jax
import jax.numpy as jnp
from jax import lax
from jax.experimental import pallas as pl
from jax.experimental.pallas import tpu as pltpu
from jax.experimental.pallas import tpu_sc as plsc

B, S, D = 2, 2048, 768
E, F = 16, 1024
N = B * S

SB = 256
NSB = S // SB
G = B * NSB
TB = 128
NB = N // TB + E
NP = NB * TB

_NC, _NS = 2, 16
NW = _NC * _NS
TPW = N // NW
CHK = 64
NCH = TPW // CHK


def _mixer_body(x_ref, wf_ref, bf_ref, wv_ref, bv_ref, wo_ref, bo_ref,
                wg_ref, bg_ref, y_ref, pos_ref, m2_ref,
                carry, sel_s, lr_s, pc_s):
    g = pl.program_id(0)

    @pl.when(g < G)
    def _mix():
        x = x_ref[0]
        f = jax.nn.sigmoid(
            jnp.dot(x, wf_ref[...], preferred_element_type=jnp.float32)
            + bf_ref[...])
        v = (jnp.dot(x, wv_ref[...], preferred_element_type=jnp.float32)
             + bv_ref[...])
        a = f
        b = (1.0 - f) * v
        k = 1
        while k < SB:
            a_sh = jnp.concatenate(
                [jnp.ones((k, D), jnp.float32), a[:-k]], axis=0)
            b_sh = jnp.concatenate(
                [jnp.zeros((k, D), jnp.float32), b[:-k]], axis=0)
            b = b_sh * a + b
            a = a_sh * a
            k *= 2

        @pl.when(g % NSB == 0)
        def _():
            carry[...] = jnp.zeros_like(carry)

        h = b + a * carry[...]
        carry[...] = h[SB - 1:SB, :]
        y = (x + jnp.dot(h, wo_ref[...], preferred_element_type=jnp.float32)
             + bo_ref[...])
        y_ref[0] = y.astype(jnp.bfloat16)

        logits = (jnp.dot(y, wg_ref[...], preferred_element_type=jnp.float32)
                  + bg_ref[...])
        m = jnp.max(logits, axis=1, keepdims=True)
        ex = jnp.exp(logits - m)
        p = ex / jnp.sum(ex, axis=1, keepdims=True)
        pm = jnp.max(p, axis=1, keepdims=True)
        eidx = lax.broadcasted_iota(jnp.int32, (SB, E), 1)
        sel = jnp.min(jnp.where(p == pm, eidx, E), axis=1, keepdims=True)

        oh = (sel == eidx).astype(jnp.float32)
        ti = lax.broadcasted_iota(jnp.int32, (SB, SB), 0)
        tj = lax.broadcasted_iota(jnp.int32, (SB, SB), 1)
        trist = (tj < ti).astype(jnp.float32)
        cum = jnp.dot(trist, oh, preferred_element_type=jnp.float32)
        lrank = jnp.sum(oh * cum, axis=1, keepdims=True)
        base = pl.multiple_of(g * SB, SB)
        sel_s[pl.ds(base, SB)] = sel
        lr_s[pl.ds(base, SB)] = lrank
        pc_s[pl.ds(g, 1), :] = jnp.sum(oh, axis=0, keepdims=True)

    @pl.when(g == G)
    def _meta():
        pc = pc_s[...]
        counts = jnp.sum(pc, axis=0, keepdims=True)
        padded = jnp.ceil(counts * (1.0 / TB)) * TB
        i16 = lax.broadcasted_iota(jnp.int32, (E, E), 0)
        j16 = lax.broadcasted_iota(jnp.int32, (E, E), 1)
        excl = (i16 < j16).astype(jnp.float32)
        starts = jnp.dot(padded, excl, preferred_element_type=jnp.float32)
        gi = lax.broadcasted_iota(jnp.int32, (G, G), 0)
        gj = lax.broadcasted_iota(jnp.int32, (G, G), 1)
        gtri = (gj < gi).astype(jnp.float32)
        offs = (jnp.dot(gtri, pc, preferred_element_type=jnp.float32)
                + starts)
        erow = lax.broadcasted_iota(jnp.int32, (SB, E), 1)
        for gg in range(G):
            sc = sel_s[gg * SB:(gg + 1) * SB]
            ohg = (sc == erow).astype(jnp.float32)
            pos_g = (jnp.sum(ohg * offs[gg:gg + 1, :], axis=1, keepdims=True)
                     + lr_s[gg * SB:(gg + 1) * SB])
            pos_ref[gg * SB:(gg + 1) * SB] = pos_g.astype(jnp.int32)
        endb = (starts + padded) * (1.0 / TB)
        icol = lax.broadcasted_iota(jnp.int32, (128, E), 0).astype(jnp.float32)
        b2e = jnp.sum((icol >= endb).astype(jnp.float32), axis=1,
                      keepdims=True)
        b2e = jnp.minimum(b2e, float(E - 1))
        total = jnp.sum(padded) * (1.0 / TB)
        m2_ref[...] = jnp.concatenate(
            [b2e, jnp.full((128, 1), total, jnp.float32)], axis=0)


def _mixer_call(x, Wf, bf, Wv, bv, Wo, bo, Wg, bg):
    full = lambda shape: pl.BlockSpec(shape, lambda g: (0,) * len(shape))

    def xmap(g):
        gc = jnp.minimum(g, G - 1)
        return (gc // NSB, gc % NSB, 0)

    return pl.pallas_call(
        _mixer_body,
        grid=(G + 1,),
        in_specs=[
            pl.BlockSpec((1, SB, D), xmap),
            full((D, D)), full((1, D)),
            full((D, D)), full((1, D)),
            full((D, D)), full((1, D)),
            full((D, E)), full((1, E)),
        ],
        out_specs=[
            pl.BlockSpec((1, SB, D), xmap),
            pl.BlockSpec((N, 1), lambda g: (0, 0)),
            pl.BlockSpec((256, 1), lambda g: (0, 0)),
        ],
        out_shape=[
            jax.ShapeDtypeStruct((B, S, D), jnp.bfloat16),
            jax.ShapeDtypeStruct((N, 1), jnp.int32),
            jax.ShapeDtypeStruct((256, 1), jnp.float32),
        ],
        scratch_shapes=[pltpu.VMEM((1, D), jnp.float32),
                        pltpu.VMEM((N, 1), jnp.int32),
                        pltpu.VMEM((N, 1), jnp.float32),
                        pltpu.VMEM((G, E), jnp.float32)],
    )(x, Wf, bf.reshape(1, D), Wv, bv.reshape(1, D),
      Wo, bo.reshape(1, D), Wg, bg.reshape(1, E))


@functools.lru_cache(maxsize=None)
def _sc_kernels():
    mesh = plsc.VectorSubcoreMesh(core_axis_name="c", subcore_axis_name="s",
                                  num_cores=_NC, num_subcores=_NS)

    D2 = D // 2

    @functools.partial(
        pl.kernel, mesh=mesh,
        out_type=jax.ShapeDtypeStruct((NP, D2), jnp.int32),
        scratch_types=[pltpu.VMEM((TPW,), jnp.int32),
                       pltpu.VMEM((TPW, D2), jnp.int32),
                       pltpu.SemaphoreType.DMA],
    )
    def _scatter(y_hbm, pos_hbm, out_hbm, idx_v, rows_v, sem):
        wid = lax.axis_index("s") * _NC + lax.axis_index("c")
        base = wid * TPW
        pltpu.sync_copy(pos_hbm.at[pl.ds(base, TPW)], idx_v)
        pltpu.sync_copy(y_hbm.at[pl.ds(base, TPW)], rows_v)
        pltpu.async_copy(rows_v, out_hbm.at[idx_v], sem).wait()

    @functools.partial(
        pl.kernel, mesh=mesh,
        out_type=jax.ShapeDtypeStruct((N, D), jnp.float32),
        scratch_types=[pltpu.VMEM((TPW,), jnp.int32),
                       pltpu.VMEM((TPW, D), jnp.float32),
                       pltpu.SemaphoreType.DMA],
    )
    def _gather(src_hbm, pos_hbm, out_hbm, idx_v, rows_v, sem):
        wid = lax.axis_index("s") * _NC + lax.axis_index("c")
        base = wid * TPW
        pltpu.sync_copy(pos_hbm.at[pl.ds(base, TPW)], idx_v)
        pltpu.async_copy(src_hbm.at[idx_v], rows_v, sem).wait()
        pltpu.sync_copy(rows_v, out_hbm.at[pl.ds(base, TPW)])

    return _scatter, _gather


def _ffn_body(b2e_ref, nb_ref, x_ref, w1_ref, b1_ref, w2_ref, b2_ref, o_ref):
    i = pl.program_id(0)

    @pl.when(i < nb_ref[0])
    def _():
        x = x_ref[...]
        h = (jnp.dot(x, w1_ref[0].astype(jnp.bfloat16),
                     preferred_element_type=jnp.float32) + b1_ref[0])
        h = jax.nn.gelu(h)
        o_ref[...] = (jnp.dot(h.astype(jnp.bfloat16),
                              w2_ref[0].astype(jnp.bfloat16),
                              preferred_element_type=jnp.float32) + b2_ref[0])


def _ffn_call(b2e, nbu, xs, W1, b1, W2, b2):
    grid_spec = pltpu.PrefetchScalarGridSpec(
        num_scalar_prefetch=2,
        grid=(NB,),
        in_specs=[
            pl.BlockSpec((TB, D),
                         lambda i, m, n: (jnp.minimum(i, n[0] - 1), 0)),
            pl.BlockSpec((1, D, F), lambda i, m, n: (m[i], 0, 0)),
            pl.BlockSpec((1, 1, F), lambda i, m, n: (m[i], 0, 0)),
            pl.BlockSpec((1, F, D), lambda i, m, n: (m[i], 0, 0)),
            pl.BlockSpec((1, 1, D), lambda i, m, n: (m[i], 0, 0)),
        ],
        out_specs=pl.BlockSpec((TB, D),
                               lambda i, m, n: (jnp.minimum(i, n[0] - 1), 0)),
    )
    return pl.pallas_call(
        _ffn_body,
        grid_spec=grid_spec,
        out_shape=jax.ShapeDtypeStruct((NP, D), jnp.float32),
    )(b2e, nbu, xs, W1, b1.reshape(E, 1, F), W2, b2.reshape(E, 1, D))


def kernel(hidden_states, Wf, bf, Wv, bv, Wo, bo, Wg, bg, W1, b1, W2, b2):
    y, posc, m2 = _mixer_call(hidden_states, Wf, bf, Wv, bv, Wo, bo, Wg, bg)
    pos = posc.reshape(N)
    b2e = m2[:NB, 0].astype(jnp.int32)
    nbu = m2[128:129, 0].astype(jnp.int32)
    sc_scatter, sc_gather = _sc_kernels()
    y32 = lax.bitcast_convert_type(y.reshape(N, D // 2, 2), jnp.int32)
    ysorted = lax.bitcast_convert_type(
        sc_scatter(y32, pos), jnp.bfloat16).reshape(NP, D)
    osorted = _ffn_call(b2e, nbu, ysorted, W1, b1, W2, b2)
    final = sc_gather(osorted, pos)
    return final.reshape(B, S, D)


# speedup vs baseline: 2.0494x; 2.0494x over previous
import functools

import jax
import jax.numpy as jnp
from jax import lax
from jax.experimental import pallas as pl
from jax.experimental.pallas import tpu as pltpu
from jax.experimental.pallas import tpu_sc as plsc

B, S, D = 2, 2048, 768
E, F = 16, 1024
N = B * S

SB = 256
NSB = S // SB
G = B * NSB
TB = 128
NB = N // TB + E
NP = NB * TB

_NC, _NS = 2, 16
NW = _NC * _NS
TPW = N // NW
CHK = 64
NCH = TPW // CHK


def _mixer_body(x_ref, wf_ref, bf_ref, wv_ref, bv_ref, wo_ref, bo_ref,
                wg_ref, bg_ref, y_ref, pos_ref, m2_ref,
                carry, sel_s, lr_s, pc_s):
    g = pl.program_id(0)

    @pl.when(g < G)
    def _mix():
        x = x_ref[0]
        f = jax.nn.sigmoid(
            jnp.dot(x, wf_ref[...], preferred_element_type=jnp.float32)
            + bf_ref[...])
        v = (jnp.dot(x, wv_ref[...], preferred_element_type=jnp.float32)
             + bv_ref[...])
        CL = 16
        NC3 = SB // CL
        a3 = f.reshape(NC3, CL, D)
        b3 = ((1.0 - f) * v).reshape(NC3, CL, D)
        k = 1
        while k < CL:
            a_sh = jnp.concatenate(
                [jnp.ones((NC3, k, D), jnp.float32), a3[:, :-k, :]], axis=1)
            b_sh = jnp.concatenate(
                [jnp.zeros((NC3, k, D), jnp.float32), b3[:, :-k, :]], axis=1)
            b3 = b_sh * a3 + b3
            a3 = a_sh * a3
            k *= 2
        ac = a3[:, CL - 1, :]
        bc = b3[:, CL - 1, :]
        k = 1
        while k < NC3:
            acs = jnp.concatenate(
                [jnp.ones((k, D), jnp.float32), ac[:-k]], axis=0)
            bcs = jnp.concatenate(
                [jnp.zeros((k, D), jnp.float32), bc[:-k]], axis=0)
            bc = bcs * ac + bc
            ac = acs * ac
            k *= 2

        @pl.when(g % NSB == 0)
        def _():
            carry[...] = jnp.zeros_like(carry)

        bc_ex = jnp.concatenate(
            [jnp.zeros((1, D), jnp.float32), bc[:-1]], axis=0)
        ac_ex = jnp.concatenate(
            [jnp.ones((1, D), jnp.float32), ac[:-1]], axis=0)
        hrow = bc_ex + ac_ex * carry[...]
        h = (b3 + a3 * hrow[:, None, :]).reshape(SB, D)
        carry[...] = h[SB - 1:SB, :]
        y = (x + jnp.dot(h, wo_ref[...], preferred_element_type=jnp.float32)
             + bo_ref[...])
        y_ref[0] = y

        logits = (jnp.dot(y, wg_ref[...], preferred_element_type=jnp.float32)
                  + bg_ref[...])
        m = jnp.max(logits, axis=1, keepdims=True)
        ex = jnp.exp(logits - m)
        p = ex / jnp.sum(ex, axis=1, keepdims=True)
        pm = jnp.max(p, axis=1, keepdims=True)
        eidx = lax.broadcasted_iota(jnp.int32, (SB, E), 1)
        sel = jnp.min(jnp.where(p == pm, eidx, E), axis=1, keepdims=True)

        oh = (sel == eidx).astype(jnp.float32)
        ti = lax.broadcasted_iota(jnp.int32, (SB, SB), 0)
        tj = lax.broadcasted_iota(jnp.int32, (SB, SB), 1)
        trist = (tj < ti).astype(jnp.float32)
        cum = jnp.dot(trist, oh, preferred_element_type=jnp.float32)
        lrank = jnp.sum(oh * cum, axis=1, keepdims=True)
        base = pl.multiple_of(g * SB, SB)
        sel_s[pl.ds(base, SB)] = sel
        lr_s[pl.ds(base, SB)] = lrank
        pc_s[pl.ds(g, 1), :] = jnp.sum(oh, axis=0, keepdims=True)

    @pl.when(g == G)
    def _meta():
        pc = pc_s[...]
        counts = jnp.sum(pc, axis=0, keepdims=True)
        padded = jnp.ceil(counts * (1.0 / TB)) * TB
        i16 = lax.broadcasted_iota(jnp.int32, (E, E), 0)
        j16 = lax.broadcasted_iota(jnp.int32, (E, E), 1)
        excl = (i16 < j16).astype(jnp.float32)
        starts = jnp.dot(padded, excl, preferred_element_type=jnp.float32)
        gi = lax.broadcasted_iota(jnp.int32, (G, G), 0)
        gj = lax.broadcasted_iota(jnp.int32, (G, G), 1)
        gtri = (gj < gi).astype(jnp.float32)
        offs = (jnp.dot(gtri, pc, preferred_element_type=jnp.float32)
                + starts)
        erow = lax.broadcasted_iota(jnp.int32, (SB, E), 1)
        for gg in range(G):
            sc = sel_s[gg * SB:(gg + 1) * SB]
            ohg = (sc == erow).astype(jnp.float32)
            pos_g = (jnp.sum(ohg * offs[gg:gg + 1, :], axis=1, keepdims=True)
                     + lr_s[gg * SB:(gg + 1) * SB])
            pos_ref[gg * SB:(gg + 1) * SB] = pos_g.astype(jnp.int32)
        endb = (starts + padded) * (1.0 / TB)
        icol = lax.broadcasted_iota(jnp.int32, (128, E), 0).astype(jnp.float32)
        b2e = jnp.sum((icol >= endb).astype(jnp.float32), axis=1,
                      keepdims=True)
        b2e = jnp.minimum(b2e, float(E - 1))
        total = jnp.sum(padded) * (1.0 / TB)
        m2_ref[...] = jnp.concatenate(
            [b2e, jnp.full((128, 1), total, jnp.float32)], axis=0)


def _mixer_call(x, Wf, bf, Wv, bv, Wo, bo, Wg, bg):
    full = lambda shape: pl.BlockSpec(shape, lambda g: (0,) * len(shape))

    def xmap(g):
        gc = jnp.minimum(g, G - 1)
        return (gc // NSB, gc % NSB, 0)

    return pl.pallas_call(
        _mixer_body,
        grid=(G + 1,),
        in_specs=[
            pl.BlockSpec((1, SB, D), xmap),
            full((D, D)), full((1, D)),
            full((D, D)), full((1, D)),
            full((D, D)), full((1, D)),
            full((D, E)), full((1, E)),
        ],
        out_specs=[
            pl.BlockSpec((1, SB, D), xmap),
            pl.BlockSpec((N, 1), lambda g: (0, 0)),
            pl.BlockSpec((256, 1), lambda g: (0, 0)),
        ],
        out_shape=[
            jax.ShapeDtypeStruct((B, S, D), jnp.float32),
            jax.ShapeDtypeStruct((N, 1), jnp.int32),
            jax.ShapeDtypeStruct((256, 1), jnp.float32),
        ],
        scratch_shapes=[pltpu.VMEM((1, D), jnp.float32),
                        pltpu.VMEM((N, 1), jnp.int32),
                        pltpu.VMEM((N, 1), jnp.float32),
                        pltpu.VMEM((G, E), jnp.float32)],
    )(x, Wf, bf.reshape(1, D), Wv, bv.reshape(1, D),
      Wo, bo.reshape(1, D), Wg, bg.reshape(1, E))


@functools.lru_cache(maxsize=None)
def _sc_kernels():
    mesh = plsc.VectorSubcoreMesh(core_axis_name="c", subcore_axis_name="s",
                                  num_cores=_NC, num_subcores=_NS)

    @functools.partial(
        pl.kernel, mesh=mesh,
        out_type=jax.ShapeDtypeStruct((NP, D), jnp.float32),
        scratch_types=[pltpu.VMEM((TPW,), jnp.int32),
                       pltpu.VMEM((TPW, D), jnp.float32),
                       pltpu.SemaphoreType.DMA],
    )
    def _scatter(y_hbm, pos_hbm, out_hbm, idx_v, rows_v, sem):
        wid = lax.axis_index("s") * _NC + lax.axis_index("c")
        base = wid * TPW
        pltpu.sync_copy(pos_hbm.at[pl.ds(base, TPW)], idx_v)
        pltpu.sync_copy(y_hbm.at[pl.ds(base, TPW)], rows_v)
        pltpu.async_copy(rows_v, out_hbm.at[idx_v], sem).wait()

    @functools.partial(
        pl.kernel, mesh=mesh,
        out_type=jax.ShapeDtypeStruct((N, D), jnp.float32),
        scratch_types=[pltpu.VMEM((TPW,), jnp.int32),
                       pltpu.VMEM((TPW, D), jnp.float32),
                       pltpu.SemaphoreType.DMA],
    )
    def _gather(src_hbm, pos_hbm, out_hbm, idx_v, rows_v, sem):
        wid = lax.axis_index("s") * _NC + lax.axis_index("c")
        base = wid * TPW
        pltpu.sync_copy(pos_hbm.at[pl.ds(base, TPW)], idx_v)
        pltpu.async_copy(src_hbm.at[idx_v], rows_v, sem).wait()
        pltpu.sync_copy(rows_v, out_hbm.at[pl.ds(base, TPW)])

    return _scatter, _gather


def _ffn_body(b2e_ref, nb_ref, x_ref, w1_ref, b1_ref, w2_ref, b2_ref, o_ref):
    i = pl.program_id(0)

    @pl.when(i < nb_ref[0])
    def _():
        x = x_ref[...]
        h = jnp.dot(x, w1_ref[0], preferred_element_type=jnp.float32) + b1_ref[0]
        h = jax.nn.gelu(h)
        o_ref[...] = (jnp.dot(h, w2_ref[0], preferred_element_type=jnp.float32)
                      + b2_ref[0])


def _ffn_call(b2e, nbu, xs, W1, b1, W2, b2):
    grid_spec = pltpu.PrefetchScalarGridSpec(
        num_scalar_prefetch=2,
        grid=(NB,),
        in_specs=[
            pl.BlockSpec((TB, D),
                         lambda i, m, n: (jnp.minimum(i, n[0] - 1), 0)),
            pl.BlockSpec((1, D, F), lambda i, m, n: (m[i], 0, 0)),
            pl.BlockSpec((1, 1, F), lambda i, m, n: (m[i], 0, 0)),
            pl.BlockSpec((1, F, D), lambda i, m, n: (m[i], 0, 0)),
            pl.BlockSpec((1, 1, D), lambda i, m, n: (m[i], 0, 0)),
        ],
        out_specs=pl.BlockSpec((TB, D),
                               lambda i, m, n: (jnp.minimum(i, n[0] - 1), 0)),
    )
    return pl.pallas_call(
        _ffn_body,
        grid_spec=grid_spec,
        out_shape=jax.ShapeDtypeStruct((NP, D), jnp.float32),
    )(b2e, nbu, xs, W1, b1.reshape(E, 1, F), W2, b2.reshape(E, 1, D))


def kernel(hidden_states, Wf, bf, Wv, bv, Wo, bo, Wg, bg, W1, b1, W2, b2):
    y, posc, m2 = _mixer_call(hidden_states, Wf, bf, Wv, bv, Wo, bo, Wg, bg)
    pos = posc.reshape(N)
    b2e = m2[:NB, 0].astype(jnp.int32)
    nbu = m2[128:129, 0].astype(jnp.int32)
    sc_scatter, sc_gather = _sc_kernels()
    ysorted = sc_scatter(y.reshape(N, D), pos)
    osorted = _ffn_call(b2e, nbu, ysorted, W1, b1, W2, b2)
    final = sc_gather(osorted, pos)
    return final.reshape(B, S, D)


# speedup vs baseline: 2.0796x; 1.0147x over previous
import functools

import jax
import jax.numpy as jnp
from jax import lax
from jax.experimental import pallas as pl
from jax.experimental.pallas import tpu as pltpu
from jax.experimental.pallas import tpu_sc as plsc

B, S, D = 2, 2048, 768
E, F = 16, 1024
N = B * S

SB = 256
NSB = S // SB
G = B * NSB
TB = 128
NB = N // TB + E
NP = NB * TB

_NC, _NS = 2, 16
NW = _NC * _NS
TPW = N // NW
CHK = 64
NCH = TPW // CHK


def _mixer_body(x_ref, wf_ref, bf_ref, wv_ref, bv_ref, wo_ref, bo_ref,
                wg_ref, bg_ref, y_ref, pos_ref, m2_ref,
                carry, sel_s, lr_s, pc_s):
    g = pl.program_id(0)

    @pl.when(g < G)
    def _mix():
        x = x_ref[0]
        f = jax.nn.sigmoid(
            jnp.dot(x, wf_ref[...], preferred_element_type=jnp.float32)
            + bf_ref[...])
        v = (jnp.dot(x, wv_ref[...], preferred_element_type=jnp.float32)
             + bv_ref[...])
        CL = 16
        NC3 = SB // CL
        a3 = f.reshape(NC3, CL, D)
        b3 = ((1.0 - f) * v).reshape(NC3, CL, D)
        k = 1
        while k < CL:
            a_sh = jnp.concatenate(
                [jnp.ones((NC3, k, D), jnp.float32), a3[:, :-k, :]], axis=1)
            b_sh = jnp.concatenate(
                [jnp.zeros((NC3, k, D), jnp.float32), b3[:, :-k, :]], axis=1)
            b3 = b_sh * a3 + b3
            a3 = a_sh * a3
            k *= 2
        ac = a3[:, CL - 1, :]
        bc = b3[:, CL - 1, :]
        k = 1
        while k < NC3:
            acs = jnp.concatenate(
                [jnp.ones((k, D), jnp.float32), ac[:-k]], axis=0)
            bcs = jnp.concatenate(
                [jnp.zeros((k, D), jnp.float32), bc[:-k]], axis=0)
            bc = bcs * ac + bc
            ac = acs * ac
            k *= 2

        @pl.when(g % NSB == 0)
        def _():
            carry[...] = jnp.zeros_like(carry)

        bc_ex = jnp.concatenate(
            [jnp.zeros((1, D), jnp.float32), bc[:-1]], axis=0)
        ac_ex = jnp.concatenate(
            [jnp.ones((1, D), jnp.float32), ac[:-1]], axis=0)
        hrow = bc_ex + ac_ex * carry[...]
        h = (b3 + a3 * hrow[:, None, :]).reshape(SB, D)
        carry[...] = h[SB - 1:SB, :]
        y = (x + jnp.dot(h, wo_ref[...], preferred_element_type=jnp.float32)
             + bo_ref[...])
        y_ref[0] = y

        logits = (jnp.dot(y, wg_ref[...], preferred_element_type=jnp.float32)
                  + bg_ref[...])
        m = jnp.max(logits, axis=1, keepdims=True)
        eidx = lax.broadcasted_iota(jnp.int32, (SB, E), 1)
        sel = jnp.min(jnp.where(logits == m, eidx, E), axis=1, keepdims=True)

        oh = (sel == eidx).astype(jnp.float32)
        ti = lax.broadcasted_iota(jnp.int32, (SB, SB), 0)
        tj = lax.broadcasted_iota(jnp.int32, (SB, SB), 1)
        trist = (tj < ti).astype(jnp.float32)
        cum = jnp.dot(trist, oh, preferred_element_type=jnp.float32)
        lrank = jnp.sum(oh * cum, axis=1, keepdims=True)
        base = pl.multiple_of(g * SB, SB)
        sel_s[pl.ds(base, SB)] = sel
        lr_s[pl.ds(base, SB)] = lrank
        pc_s[pl.ds(g, 1), :] = jnp.sum(oh, axis=0, keepdims=True)

    @pl.when(g == G)
    def _meta():
        pc = pc_s[...]
        counts = jnp.sum(pc, axis=0, keepdims=True)
        padded = jnp.ceil(counts * (1.0 / TB)) * TB
        i16 = lax.broadcasted_iota(jnp.int32, (E, E), 0)
        j16 = lax.broadcasted_iota(jnp.int32, (E, E), 1)
        excl = (i16 < j16).astype(jnp.float32)
        starts = jnp.dot(padded, excl, preferred_element_type=jnp.float32)
        gi = lax.broadcasted_iota(jnp.int32, (G, G), 0)
        gj = lax.broadcasted_iota(jnp.int32, (G, G), 1)
        gtri = (gj < gi).astype(jnp.float32)
        offs = (jnp.dot(gtri, pc, preferred_element_type=jnp.float32)
                + starts)
        erow = lax.broadcasted_iota(jnp.int32, (SB, E), 1)
        for gg in range(G):
            sc = sel_s[gg * SB:(gg + 1) * SB]
            ohg = (sc == erow).astype(jnp.float32)
            pos_g = (jnp.sum(ohg * offs[gg:gg + 1, :], axis=1, keepdims=True)
                     + lr_s[gg * SB:(gg + 1) * SB])
            pos_ref[gg * SB:(gg + 1) * SB] = pos_g.astype(jnp.int32)
        endb = (starts + padded) * (1.0 / TB)
        icol = lax.broadcasted_iota(jnp.int32, (128, E), 0).astype(jnp.float32)
        b2e = jnp.sum((icol >= endb).astype(jnp.float32), axis=1,
                      keepdims=True)
        b2e = jnp.minimum(b2e, float(E - 1))
        total = jnp.sum(padded) * (1.0 / TB)
        m2_ref[...] = jnp.concatenate(
            [b2e, jnp.full((128, 1), total, jnp.float32)], axis=0)


def _mixer_call(x, Wf, bf, Wv, bv, Wo, bo, Wg, bg):
    full = lambda shape: pl.BlockSpec(shape, lambda g: (0,) * len(shape))

    def xmap(g):
        gc = jnp.minimum(g, G - 1)
        return (gc // NSB, gc % NSB, 0)

    return pl.pallas_call(
        _mixer_body,
        grid=(G + 1,),
        in_specs=[
            pl.BlockSpec((1, SB, D), xmap),
            full((D, D)), full((1, D)),
            full((D, D)), full((1, D)),
            full((D, D)), full((1, D)),
            full((D, E)), full((1, E)),
        ],
        out_specs=[
            pl.BlockSpec((1, SB, D), xmap),
            pl.BlockSpec((N, 1), lambda g: (0, 0)),
            pl.BlockSpec((256, 1), lambda g: (0, 0)),
        ],
        out_shape=[
            jax.ShapeDtypeStruct((B, S, D), jnp.float32),
            jax.ShapeDtypeStruct((N, 1), jnp.int32),
            jax.ShapeDtypeStruct((256, 1), jnp.float32),
        ],
        scratch_shapes=[pltpu.VMEM((1, D), jnp.float32),
                        pltpu.VMEM((N, 1), jnp.int32),
                        pltpu.VMEM((N, 1), jnp.float32),
                        pltpu.VMEM((G, E), jnp.float32)],
    )(x, Wf, bf.reshape(1, D), Wv, bv.reshape(1, D),
      Wo, bo.reshape(1, D), Wg, bg.reshape(1, E))


@functools.lru_cache(maxsize=None)
def _sc_kernels():
    mesh = plsc.VectorSubcoreMesh(core_axis_name="c", subcore_axis_name="s",
                                  num_cores=_NC, num_subcores=_NS)

    @functools.partial(
        pl.kernel, mesh=mesh,
        out_type=jax.ShapeDtypeStruct((NP, D), jnp.float32),
        scratch_types=[pltpu.VMEM((TPW,), jnp.int32),
                       pltpu.VMEM((TPW, D), jnp.float32),
                       pltpu.SemaphoreType.DMA],
    )
    def _scatter(y_hbm, pos_hbm, out_hbm, idx_v, rows_v, sem):
        wid = lax.axis_index("s") * _NC + lax.axis_index("c")
        base = wid * TPW
        pltpu.sync_copy(pos_hbm.at[pl.ds(base, TPW)], idx_v)
        pltpu.sync_copy(y_hbm.at[pl.ds(base, TPW)], rows_v)
        pltpu.async_copy(rows_v, out_hbm.at[idx_v], sem).wait()

    @functools.partial(
        pl.kernel, mesh=mesh,
        out_type=jax.ShapeDtypeStruct((N, D), jnp.float32),
        scratch_types=[pltpu.VMEM((TPW,), jnp.int32),
                       pltpu.VMEM((TPW, D), jnp.float32),
                       pltpu.SemaphoreType.DMA],
    )
    def _gather(src_hbm, pos_hbm, out_hbm, idx_v, rows_v, sem):
        wid = lax.axis_index("s") * _NC + lax.axis_index("c")
        base = wid * TPW
        pltpu.sync_copy(pos_hbm.at[pl.ds(base, TPW)], idx_v)
        pltpu.async_copy(src_hbm.at[idx_v], rows_v, sem).wait()
        pltpu.sync_copy(rows_v, out_hbm.at[pl.ds(base, TPW)])

    return _scatter, _gather


def _ffn_body(b2e_ref, nb_ref, x_ref, w1_ref, b1_ref, w2_ref, b2_ref, o_ref):
    i = pl.program_id(0)

    @pl.when(i < nb_ref[0])
    def _():
        x = x_ref[...]
        h = jnp.dot(x, w1_ref[0], preferred_element_type=jnp.float32) + b1_ref[0]
        h = jax.nn.gelu(h)
        o_ref[...] = (jnp.dot(h, w2_ref[0], preferred_element_type=jnp.float32)
                      + b2_ref[0])


def _ffn_call(b2e, nbu, xs, W1, b1, W2, b2):
    grid_spec = pltpu.PrefetchScalarGridSpec(
        num_scalar_prefetch=2,
        grid=(NB,),
        in_specs=[
            pl.BlockSpec((TB, D),
                         lambda i, m, n: (jnp.minimum(i, n[0] - 1), 0)),
            pl.BlockSpec((1, D, F), lambda i, m, n: (m[i], 0, 0)),
            pl.BlockSpec((1, 1, F), lambda i, m, n: (m[i], 0, 0)),
            pl.BlockSpec((1, F, D), lambda i, m, n: (m[i], 0, 0)),
            pl.BlockSpec((1, 1, D), lambda i, m, n: (m[i], 0, 0)),
        ],
        out_specs=pl.BlockSpec((TB, D),
                               lambda i, m, n: (jnp.minimum(i, n[0] - 1), 0)),
    )
    return pl.pallas_call(
        _ffn_body,
        grid_spec=grid_spec,
        out_shape=jax.ShapeDtypeStruct((NP, D), jnp.float32),
    )(b2e, nbu, xs, W1, b1.reshape(E, 1, F), W2, b2.reshape(E, 1, D))


def kernel(hidden_states, Wf, bf, Wv, bv, Wo, bo, Wg, bg, W1, b1, W2, b2):
    y, posc, m2 = _mixer_call(hidden_states, Wf, bf, Wv, bv, Wo, bo, Wg, bg)
    pos = posc.reshape(N)
    b2e = m2[:NB, 0].astype(jnp.int32)
    nbu = m2[128:129, 0].astype(jnp.int32)
    sc_scatter, sc_gather = _sc_kernels()
    ysorted = sc_scatter(y.reshape(N, D), pos)
    osorted = _ffn_call(b2e, nbu, ysorted, W1, b1, W2, b2)
    final = sc_gather(osorted, pos)
    return final.reshape(B, S, D)


# speedup vs baseline: 2.3007x; 1.1064x over previous
import functools

import jax
import jax.numpy as jnp
from jax import lax
from jax.experimental import pallas as pl
from jax.experimental.pallas import tpu as pltpu
from jax.experimental.pallas import tpu_sc as plsc

B, S, D = 2, 2048, 768
E, F = 16, 1024
N = B * S

SB = 256
NSB = S // SB
G = B * NSB
TB = 256
NB = N // TB + E
NP = NB * TB

_NC, _NS = 2, 16
NW = _NC * _NS
TPW = N // NW
CHK = 64
NCH = TPW // CHK


def _mixer_body(x_ref, wf_ref, bf_ref, wv_ref, bv_ref, wo_ref, bo_ref,
                wg_ref, bg_ref, y_ref, pos_ref, m2_ref,
                carry, sel_s, lr_s, pc_s):
    g = pl.program_id(0)

    @pl.when(g < G)
    def _mix():
        x = x_ref[0]
        f = jax.nn.sigmoid(
            jnp.dot(x, wf_ref[...], preferred_element_type=jnp.float32)
            + bf_ref[...])
        v = (jnp.dot(x, wv_ref[...], preferred_element_type=jnp.float32)
             + bv_ref[...])
        CL = 16
        NC3 = SB // CL
        a3 = f.reshape(NC3, CL, D)
        b3 = ((1.0 - f) * v).reshape(NC3, CL, D)
        k = 1
        while k < CL:
            a_sh = jnp.concatenate(
                [jnp.ones((NC3, k, D), jnp.float32), a3[:, :-k, :]], axis=1)
            b_sh = jnp.concatenate(
                [jnp.zeros((NC3, k, D), jnp.float32), b3[:, :-k, :]], axis=1)
            b3 = b_sh * a3 + b3
            a3 = a_sh * a3
            k *= 2
        ac = a3[:, CL - 1, :]
        bc = b3[:, CL - 1, :]
        k = 1
        while k < NC3:
            acs = jnp.concatenate(
                [jnp.ones((k, D), jnp.float32), ac[:-k]], axis=0)
            bcs = jnp.concatenate(
                [jnp.zeros((k, D), jnp.float32), bc[:-k]], axis=0)
            bc = bcs * ac + bc
            ac = acs * ac
            k *= 2

        @pl.when(g % NSB == 0)
        def _():
            carry[...] = jnp.zeros_like(carry)

        bc_ex = jnp.concatenate(
            [jnp.zeros((1, D), jnp.float32), bc[:-1]], axis=0)
        ac_ex = jnp.concatenate(
            [jnp.ones((1, D), jnp.float32), ac[:-1]], axis=0)
        hrow = bc_ex + ac_ex * carry[...]
        h = (b3 + a3 * hrow[:, None, :]).reshape(SB, D)
        carry[...] = h[SB - 1:SB, :]
        y = (x + jnp.dot(h, wo_ref[...], preferred_element_type=jnp.float32)
             + bo_ref[...])
        y_ref[0] = y

        logits = (jnp.dot(y, wg_ref[...], preferred_element_type=jnp.float32)
                  + bg_ref[...])
        m = jnp.max(logits, axis=1, keepdims=True)
        eidx = lax.broadcasted_iota(jnp.int32, (SB, E), 1)
        sel = jnp.min(jnp.where(logits == m, eidx, E), axis=1, keepdims=True)

        oh = (sel == eidx).astype(jnp.float32)
        ti = lax.broadcasted_iota(jnp.int32, (SB, SB), 0)
        tj = lax.broadcasted_iota(jnp.int32, (SB, SB), 1)
        trist = (tj < ti).astype(jnp.float32)
        cum = jnp.dot(trist, oh, preferred_element_type=jnp.float32)
        lrank = jnp.sum(oh * cum, axis=1, keepdims=True)
        base = pl.multiple_of(g * SB, SB)
        sel_s[pl.ds(base, SB)] = sel
        lr_s[pl.ds(base, SB)] = lrank
        pc_s[pl.ds(g, 1), :] = jnp.sum(oh, axis=0, keepdims=True)

    @pl.when(g == G)
    def _meta():
        pc = pc_s[...]
        counts = jnp.sum(pc, axis=0, keepdims=True)
        padded = jnp.ceil(counts * (1.0 / TB)) * TB
        i16 = lax.broadcasted_iota(jnp.int32, (E, E), 0)
        j16 = lax.broadcasted_iota(jnp.int32, (E, E), 1)
        excl = (i16 < j16).astype(jnp.float32)
        starts = jnp.dot(padded, excl, preferred_element_type=jnp.float32)
        gi = lax.broadcasted_iota(jnp.int32, (G, G), 0)
        gj = lax.broadcasted_iota(jnp.int32, (G, G), 1)
        gtri = (gj < gi).astype(jnp.float32)
        offs = (jnp.dot(gtri, pc, preferred_element_type=jnp.float32)
                + starts)
        erow = lax.broadcasted_iota(jnp.int32, (SB, E), 1)
        for gg in range(G):
            sc = sel_s[gg * SB:(gg + 1) * SB]
            ohg = (sc == erow).astype(jnp.float32)
            pos_g = (jnp.sum(ohg * offs[gg:gg + 1, :], axis=1, keepdims=True)
                     + lr_s[gg * SB:(gg + 1) * SB])
            pos_ref[gg * SB:(gg + 1) * SB] = pos_g.astype(jnp.int32)
        endb = (starts + padded) * (1.0 / TB)
        icol = lax.broadcasted_iota(jnp.int32, (128, E), 0).astype(jnp.float32)
        b2e = jnp.sum((icol >= endb).astype(jnp.float32), axis=1,
                      keepdims=True)
        b2e = jnp.minimum(b2e, float(E - 1))
        total = jnp.sum(padded) * (1.0 / TB)
        m2_ref[...] = jnp.concatenate(
            [b2e, jnp.full((128, 1), total, jnp.float32)], axis=0)


def _mixer_call(x, Wf, bf, Wv, bv, Wo, bo, Wg, bg):
    full = lambda shape: pl.BlockSpec(shape, lambda g: (0,) * len(shape))

    def xmap(g):
        gc = jnp.minimum(g, G - 1)
        return (gc // NSB, gc % NSB, 0)

    return pl.pallas_call(
        _mixer_body,
        grid=(G + 1,),
        in_specs=[
            pl.BlockSpec((1, SB, D), xmap),
            full((D, D)), full((1, D)),
            full((D, D)), full((1, D)),
            full((D, D)), full((1, D)),
            full((D, E)), full((1, E)),
        ],
        out_specs=[
            pl.BlockSpec((1, SB, D), xmap),
            pl.BlockSpec((N, 1), lambda g: (0, 0)),
            pl.BlockSpec((256, 1), lambda g: (0, 0)),
        ],
        out_shape=[
            jax.ShapeDtypeStruct((B, S, D), jnp.float32),
            jax.ShapeDtypeStruct((N, 1), jnp.int32),
            jax.ShapeDtypeStruct((256, 1), jnp.float32),
        ],
        scratch_shapes=[pltpu.VMEM((1, D), jnp.float32),
                        pltpu.VMEM((N, 1), jnp.int32),
                        pltpu.VMEM((N, 1), jnp.float32),
                        pltpu.VMEM((G, E), jnp.float32)],
    )(x, Wf, bf.reshape(1, D), Wv, bv.reshape(1, D),
      Wo, bo.reshape(1, D), Wg, bg.reshape(1, E))


@functools.lru_cache(maxsize=None)
def _sc_kernels():
    mesh = plsc.VectorSubcoreMesh(core_axis_name="c", subcore_axis_name="s",
                                  num_cores=_NC, num_subcores=_NS)

    @functools.partial(
        pl.kernel, mesh=mesh,
        out_type=jax.ShapeDtypeStruct((NP, D), jnp.float32),
        scratch_types=[pltpu.VMEM((TPW,), jnp.int32),
                       pltpu.VMEM((TPW, D), jnp.float32),
                       pltpu.SemaphoreType.DMA],
    )
    def _scatter(y_hbm, pos_hbm, out_hbm, idx_v, rows_v, sem):
        wid = lax.axis_index("s") * _NC + lax.axis_index("c")
        base = wid * TPW
        pltpu.sync_copy(pos_hbm.at[pl.ds(base, TPW)], idx_v)
        pltpu.sync_copy(y_hbm.at[pl.ds(base, TPW)], rows_v)
        pltpu.async_copy(rows_v, out_hbm.at[idx_v], sem).wait()

    @functools.partial(
        pl.kernel, mesh=mesh,
        out_type=jax.ShapeDtypeStruct((N, D), jnp.float32),
        scratch_types=[pltpu.VMEM((TPW,), jnp.int32),
                       pltpu.VMEM((TPW, D), jnp.float32),
                       pltpu.SemaphoreType.DMA],
    )
    def _gather(src_hbm, pos_hbm, out_hbm, idx_v, rows_v, sem):
        wid = lax.axis_index("s") * _NC + lax.axis_index("c")
        base = wid * TPW
        pltpu.sync_copy(pos_hbm.at[pl.ds(base, TPW)], idx_v)
        pltpu.async_copy(src_hbm.at[idx_v], rows_v, sem).wait()
        pltpu.sync_copy(rows_v, out_hbm.at[pl.ds(base, TPW)])

    return _scatter, _gather


def _ffn_body(b2e_ref, nb_ref, x_ref, w1_ref, b1_ref, w2_ref, b2_ref, o_ref):
    i = pl.program_id(0)

    @pl.when(i < nb_ref[0])
    def _():
        x = x_ref[...]
        h = jnp.dot(x, w1_ref[0], preferred_element_type=jnp.float32) + b1_ref[0]
        h = jax.nn.gelu(h)
        o_ref[...] = (jnp.dot(h, w2_ref[0], preferred_element_type=jnp.float32)
                      + b2_ref[0])


def _ffn_call(b2e, nbu, xs, W1, b1, W2, b2):
    grid_spec = pltpu.PrefetchScalarGridSpec(
        num_scalar_prefetch=2,
        grid=(NB,),
        in_specs=[
            pl.BlockSpec((TB, D),
                         lambda i, m, n: (jnp.minimum(i, n[0] - 1), 0)),
            pl.BlockSpec((1, D, F), lambda i, m, n: (m[i], 0, 0)),
            pl.BlockSpec((1, 1, F), lambda i, m, n: (m[i], 0, 0)),
            pl.BlockSpec((1, F, D), lambda i, m, n: (m[i], 0, 0)),
            pl.BlockSpec((1, 1, D), lambda i, m, n: (m[i], 0, 0)),
        ],
        out_specs=pl.BlockSpec((TB, D),
                               lambda i, m, n: (jnp.minimum(i, n[0] - 1), 0)),
    )
    return pl.pallas_call(
        _ffn_body,
        grid_spec=grid_spec,
        out_shape=jax.ShapeDtypeStruct((NP, D), jnp.float32),
    )(b2e, nbu, xs, W1, b1.reshape(E, 1, F), W2, b2.reshape(E, 1, D))


def kernel(hidden_states, Wf, bf, Wv, bv, Wo, bo, Wg, bg, W1, b1, W2, b2):
    y, posc, m2 = _mixer_call(hidden_states, Wf, bf, Wv, bv, Wo, bo, Wg, bg)
    pos = posc.reshape(N)
    b2e = m2[:NB, 0].astype(jnp.int32)
    nbu = m2[128:129, 0].astype(jnp.int32)
    sc_scatter, sc_gather = _sc_kernels()
    ysorted = sc_scatter(y.reshape(N, D), pos)
    osorted = _ffn_call(b2e, nbu, ysorted, W1, b1, W2, b2)
    final = sc_gather(osorted, pos)
    return final.reshape(B, S, D)


# speedup vs baseline: 2.5036x; 1.0882x over previous
import functools

import jax
import jax.numpy as jnp
from jax import lax
from jax.experimental import pallas as pl
from jax.experimental.pallas import tpu as pltpu
from jax.experimental.pallas import tpu_sc as plsc

B, S, D = 2, 2048, 768
E, F = 16, 1024
N = B * S

SB = 256
NSB = S // SB
G = B * NSB
TB = 320
NB = N // TB + E
NP = NB * TB

_NC, _NS = 2, 16
NW = _NC * _NS
TPW = N // NW
CHK = 64
NCH = TPW // CHK


def _mixer_body(x_ref, wf_ref, bf_ref, wv_ref, bv_ref, wo_ref, bo_ref,
                wg_ref, bg_ref, y_ref, pos_ref, m2_ref,
                carry, sel_s, lr_s, pc_s):
    g = pl.program_id(0)

    @pl.when(g < G)
    def _mix():
        x = x_ref[0]
        f = jax.nn.sigmoid(
            jnp.dot(x, wf_ref[...], preferred_element_type=jnp.float32)
            + bf_ref[...])
        v = (jnp.dot(x, wv_ref[...], preferred_element_type=jnp.float32)
             + bv_ref[...])
        CL = 16
        NC3 = SB // CL
        a3 = f.reshape(NC3, CL, D)
        b3 = ((1.0 - f) * v).reshape(NC3, CL, D)
        k = 1
        while k < CL:
            a_sh = jnp.concatenate(
                [jnp.ones((NC3, k, D), jnp.float32), a3[:, :-k, :]], axis=1)
            b_sh = jnp.concatenate(
                [jnp.zeros((NC3, k, D), jnp.float32), b3[:, :-k, :]], axis=1)
            b3 = b_sh * a3 + b3
            a3 = a_sh * a3
            k *= 2
        ac = a3[:, CL - 1, :]
        bc = b3[:, CL - 1, :]
        k = 1
        while k < NC3:
            acs = jnp.concatenate(
                [jnp.ones((k, D), jnp.float32), ac[:-k]], axis=0)
            bcs = jnp.concatenate(
                [jnp.zeros((k, D), jnp.float32), bc[:-k]], axis=0)
            bc = bcs * ac + bc
            ac = acs * ac
            k *= 2

        @pl.when(g % NSB == 0)
        def _():
            carry[...] = jnp.zeros_like(carry)

        bc_ex = jnp.concatenate(
            [jnp.zeros((1, D), jnp.float32), bc[:-1]], axis=0)
        ac_ex = jnp.concatenate(
            [jnp.ones((1, D), jnp.float32), ac[:-1]], axis=0)
        hrow = bc_ex + ac_ex * carry[...]
        h = (b3 + a3 * hrow[:, None, :]).reshape(SB, D)
        carry[...] = h[SB - 1:SB, :]
        y = (x + jnp.dot(h, wo_ref[...], preferred_element_type=jnp.float32)
             + bo_ref[...])
        y_ref[0] = y

        logits = (jnp.dot(y, wg_ref[...], preferred_element_type=jnp.float32)
                  + bg_ref[...])
        m = jnp.max(logits, axis=1, keepdims=True)
        eidx = lax.broadcasted_iota(jnp.int32, (SB, E), 1)
        sel = jnp.min(jnp.where(logits == m, eidx, E), axis=1, keepdims=True)

        oh = (sel == eidx).astype(jnp.float32)
        ti = lax.broadcasted_iota(jnp.int32, (SB, SB), 0)
        tj = lax.broadcasted_iota(jnp.int32, (SB, SB), 1)
        trist = (tj < ti).astype(jnp.float32)
        cum = jnp.dot(trist, oh, preferred_element_type=jnp.float32)
        lrank = jnp.sum(oh * cum, axis=1, keepdims=True)
        base = pl.multiple_of(g * SB, SB)
        sel_s[pl.ds(base, SB)] = sel
        lr_s[pl.ds(base, SB)] = lrank
        pc_s[pl.ds(g, 1), :] = jnp.sum(oh, axis=0, keepdims=True)

    @pl.when(g == G)
    def _meta():
        pc = pc_s[...]
        counts = jnp.sum(pc, axis=0, keepdims=True)
        padded = jnp.ceil(counts * (1.0 / TB)) * TB
        i16 = lax.broadcasted_iota(jnp.int32, (E, E), 0)
        j16 = lax.broadcasted_iota(jnp.int32, (E, E), 1)
        excl = (i16 < j16).astype(jnp.float32)
        starts = jnp.dot(padded, excl, preferred_element_type=jnp.float32)
        gi = lax.broadcasted_iota(jnp.int32, (G, G), 0)
        gj = lax.broadcasted_iota(jnp.int32, (G, G), 1)
        gtri = (gj < gi).astype(jnp.float32)
        offs = (jnp.dot(gtri, pc, preferred_element_type=jnp.float32)
                + starts)
        erow = lax.broadcasted_iota(jnp.int32, (SB, E), 1)
        for gg in range(G):
            sc = sel_s[gg * SB:(gg + 1) * SB]
            ohg = (sc == erow).astype(jnp.float32)
            pos_g = (jnp.sum(ohg * offs[gg:gg + 1, :], axis=1, keepdims=True)
                     + lr_s[gg * SB:(gg + 1) * SB])
            pos_ref[gg * SB:(gg + 1) * SB] = pos_g.astype(jnp.int32)
        endb = (starts + padded) * (1.0 / TB)
        icol = lax.broadcasted_iota(jnp.int32, (128, E), 0).astype(jnp.float32)
        b2e = jnp.sum((icol >= endb).astype(jnp.float32), axis=1,
                      keepdims=True)
        b2e = jnp.minimum(b2e, float(E - 1))
        total = jnp.sum(padded) * (1.0 / TB)
        m2_ref[...] = jnp.concatenate(
            [b2e, jnp.full((128, 1), total, jnp.float32)], axis=0)


def _mixer_call(x, Wf, bf, Wv, bv, Wo, bo, Wg, bg):
    full = lambda shape: pl.BlockSpec(shape, lambda g: (0,) * len(shape))

    def xmap(g):
        gc = jnp.minimum(g, G - 1)
        return (gc // NSB, gc % NSB, 0)

    return pl.pallas_call(
        _mixer_body,
        grid=(G + 1,),
        in_specs=[
            pl.BlockSpec((1, SB, D), xmap),
            full((D, D)), full((1, D)),
            full((D, D)), full((1, D)),
            full((D, D)), full((1, D)),
            full((D, E)), full((1, E)),
        ],
        out_specs=[
            pl.BlockSpec((1, SB, D), xmap),
            pl.BlockSpec((N, 1), lambda g: (0, 0)),
            pl.BlockSpec((256, 1), lambda g: (0, 0)),
        ],
        out_shape=[
            jax.ShapeDtypeStruct((B, S, D), jnp.float32),
            jax.ShapeDtypeStruct((N, 1), jnp.int32),
            jax.ShapeDtypeStruct((256, 1), jnp.float32),
        ],
        scratch_shapes=[pltpu.VMEM((1, D), jnp.float32),
                        pltpu.VMEM((N, 1), jnp.int32),
                        pltpu.VMEM((N, 1), jnp.float32),
                        pltpu.VMEM((G, E), jnp.float32)],
    )(x, Wf, bf.reshape(1, D), Wv, bv.reshape(1, D),
      Wo, bo.reshape(1, D), Wg, bg.reshape(1, E))


@functools.lru_cache(maxsize=None)
def _sc_kernels():
    mesh = plsc.VectorSubcoreMesh(core_axis_name="c", subcore_axis_name="s",
                                  num_cores=_NC, num_subcores=_NS)

    @functools.partial(
        pl.kernel, mesh=mesh,
        out_type=jax.ShapeDtypeStruct((NP, D), jnp.float32),
        scratch_types=[pltpu.VMEM((TPW,), jnp.int32),
                       pltpu.VMEM((TPW, D), jnp.float32),
                       pltpu.SemaphoreType.DMA],
    )
    def _scatter(y_hbm, pos_hbm, out_hbm, idx_v, rows_v, sem):
        wid = lax.axis_index("s") * _NC + lax.axis_index("c")
        base = wid * TPW
        pltpu.sync_copy(pos_hbm.at[pl.ds(base, TPW)], idx_v)
        pltpu.sync_copy(y_hbm.at[pl.ds(base, TPW)], rows_v)
        pltpu.async_copy(rows_v, out_hbm.at[idx_v], sem).wait()

    @functools.partial(
        pl.kernel, mesh=mesh,
        out_type=jax.ShapeDtypeStruct((N, D), jnp.float32),
        scratch_types=[pltpu.VMEM((TPW,), jnp.int32),
                       pltpu.VMEM((TPW, D), jnp.float32),
                       pltpu.SemaphoreType.DMA],
    )
    def _gather(src_hbm, pos_hbm, out_hbm, idx_v, rows_v, sem):
        wid = lax.axis_index("s") * _NC + lax.axis_index("c")
        base = wid * TPW
        pltpu.sync_copy(pos_hbm.at[pl.ds(base, TPW)], idx_v)
        pltpu.async_copy(src_hbm.at[idx_v], rows_v, sem).wait()
        pltpu.sync_copy(rows_v, out_hbm.at[pl.ds(base, TPW)])

    return _scatter, _gather


def _ffn_body(b2e_ref, nb_ref, x_ref, w1_ref, b1_ref, w2_ref, b2_ref, o_ref):
    i = pl.program_id(0)

    @pl.when(i < nb_ref[0])
    def _():
        x = x_ref[...]
        h = jnp.dot(x, w1_ref[0], preferred_element_type=jnp.float32) + b1_ref[0]
        h = jax.nn.gelu(h)
        o_ref[...] = (jnp.dot(h, w2_ref[0], preferred_element_type=jnp.float32)
                      + b2_ref[0])


def _ffn_call(b2e, nbu, xs, W1, b1, W2, b2):
    grid_spec = pltpu.PrefetchScalarGridSpec(
        num_scalar_prefetch=2,
        grid=(NB,),
        in_specs=[
            pl.BlockSpec((TB, D),
                         lambda i, m, n: (jnp.minimum(i, n[0] - 1), 0)),
            pl.BlockSpec((1, D, F), lambda i, m, n: (m[i], 0, 0)),
            pl.BlockSpec((1, 1, F), lambda i, m, n: (m[i], 0, 0)),
            pl.BlockSpec((1, F, D), lambda i, m, n: (m[i], 0, 0)),
            pl.BlockSpec((1, 1, D), lambda i, m, n: (m[i], 0, 0)),
        ],
        out_specs=pl.BlockSpec((TB, D),
                               lambda i, m, n: (jnp.minimum(i, n[0] - 1), 0)),
    )
    return pl.pallas_call(
        _ffn_body,
        grid_spec=grid_spec,
        out_shape=jax.ShapeDtypeStruct((NP, D), jnp.float32),
    )(b2e, nbu, xs, W1, b1.reshape(E, 1, F), W2, b2.reshape(E, 1, D))


def kernel(hidden_states, Wf, bf, Wv, bv, Wo, bo, Wg, bg, W1, b1, W2, b2):
    y, posc, m2 = _mixer_call(hidden_states, Wf, bf, Wv, bv, Wo, bo, Wg, bg)
    pos = posc.reshape(N)
    b2e = m2[:NB, 0].astype(jnp.int32)
    nbu = m2[128:129, 0].astype(jnp.int32)
    sc_scatter, sc_gather = _sc_kernels()
    ysorted = sc_scatter(y.reshape(N, D), pos)
    osorted = _ffn_call(b2e, nbu, ysorted, W1, b1, W2, b2)
    final = sc_gather(osorted, pos)
    return final.reshape(B, S, D)
